# baseline (speedup 1.0000x reference)
import functools

import jax
import jax.numpy as jnp
from jax import lax
from jax.experimental import pallas as pl
from jax.experimental.pallas import tpu as pltpu
from jax.experimental.pallas import tpu_sc as plsc

_VOCAB = 1000000
_D = 32
_B = 4096
_T = 200
_NC = 2
_NS = 16
_NW = _NC * _NS
_BG = 256
_TG = 8
_NTASK = (_B // _BG) * (_T // _TG)
_MAXK = (_NTASK + _NW - 1) // _NW
_L = 16


@functools.partial(
    pl.kernel,
    out_type=jax.ShapeDtypeStruct((_T, _D, _B), jnp.float32),
    mesh=plsc.VectorSubcoreMesh(core_axis_name="c", subcore_axis_name="s"),
    scratch_types=[
        pltpu.VMEM((_BG,), jnp.int32),
        pltpu.VMEM((_BG,), jnp.int32),
        pltpu.VMEM((_BG,), jnp.int32),
        pltpu.VMEM((_BG,), jnp.int32),
        pltpu.VMEM((2, _BG, 128), jnp.float32),
        pltpu.VMEM((2, _D, _BG), jnp.float32),
        pltpu.SemaphoreType.DMA,
        pltpu.SemaphoreType.DMA,
        pltpu.SemaphoreType.DMA,
        pltpu.SemaphoreType.DMA,
        pltpu.SemaphoreType.DMA,
        pltpu.SemaphoreType.DMA,
    ],
    compiler_params=pltpu.CompilerParams(use_tc_tiling_on_sc=True,
                                         needs_layout_passes=False),
)
def _gather_kernel(idx_hbm, table_hbm, out_hbm, idxva, idxvb, widxa, widxb,
                   rows, outb, isem0, isem1, gsem0, gsem1, wsem0, wsem1):
    gsem = (gsem0, gsem1)
    wsem = (wsem0, wsem1)
    isem = (isem0, isem1)
    idxv = (idxva, idxvb)
    widx = (widxa, widxb)
    wid = lax.axis_index("s") * _NC + lax.axis_index("c")
    iota = lax.iota(jnp.int32, _L)

    def run_task(tau):
        tg = tau % (_T // _TG)
        bg = tau // (_T // _TG)
        t0 = tg * _TG
        b0 = bg * _BG

        def start_idx(tt, ring):
            pltpu.async_copy(idx_hbm.at[t0 + tt, pl.ds(b0, _BG)],
                             idxv[ring], isem[ring])

        def wait_idx(tt, ring):
            pltpu.make_async_copy(idx_hbm.at[t0 + tt, pl.ds(b0, _BG)],
                                  idxv[ring], isem[ring]).wait()

        def compute_slab_idx(ring):
            for i in range(_BG // _L):
                v = idxv[ring][pl.ds(i * _L, _L)]
                widx[ring][pl.ds(i * _L, _L)] = lax.shift_right_logical(v, 2)

        def start_gather(ring):
            pltpu.async_copy(table_hbm.at[widx[ring]], rows.at[ring],
                             gsem[ring])

        def wait_gather(ring):
            pltpu.make_async_copy(table_hbm.at[widx[ring]], rows.at[ring],
                                  gsem[ring]).wait()

        def start_write(tt, ring):
            pltpu.async_copy(outb.at[ring],
                             out_hbm.at[t0 + tt, :, pl.ds(b0, _BG)],
                             wsem[ring])

        def wait_write(tt, ring):
            pltpu.make_async_copy(outb.at[ring],
                                  out_hbm.at[t0 + tt, :, pl.ds(b0, _BG)],
                                  wsem[ring]).wait()

        def transpose(ring):
            src = rows.at[ring]
            dst = outb.at[ring]

            def jbody(j, carry):
                o = j * _L
                v = idxv[ring][pl.ds(o, _L)]
                seg = lax.shift_left(lax.bitwise_and(v, 3), 5)
                rowi = o + iota
                for d in range(_D):
                    dst[d, pl.ds(o, _L)] = plsc.load_gather(
                        src, [rowi, seg + d])
                return carry

            lax.fori_loop(0, _BG // _L, jbody, 0)

        start_idx(0, 0)
        wait_idx(0, 0)
        compute_slab_idx(0)
        start_gather(0)
        start_idx(1, 1)
        for tt in range(_TG):
            ring = tt % 2
            wait_gather(ring)
            if tt + 1 < _TG:
                wait_idx(tt + 1, 1 - ring)
                compute_slab_idx(1 - ring)
                start_gather(1 - ring)
            if tt >= 2:
                wait_write(tt - 2, ring)
            transpose(ring)
            if tt + 2 < _TG:
                start_idx(tt + 2, ring)
            start_write(tt, ring)
        wait_write(_TG - 2, 0)
        wait_write(_TG - 1, 1)

    def body(k, carry):
        tau = wid + k * _NW

        @pl.when(tau < _NTASK)
        def _():
            run_task(tau)
        return carry

    lax.fori_loop(0, _MAXK, body, 0)


def kernel(x, weight):
    w4 = weight.reshape(_VOCAB // 4, 4 * _D)
    out = _gather_kernel(jnp.transpose(x), w4)
    return jnp.transpose(out, (2, 0, 1))


# speedup vs baseline: 1.2441x; 1.2441x over previous
import functools

import jax
import jax.numpy as jnp
from jax import lax
from jax.experimental import pallas as pl
from jax.experimental.pallas import tpu as pltpu
from jax.experimental.pallas import tpu_sc as plsc

_VOCAB = 1000000
_D = 32
_B = 4096
_T = 200
_N = _B * _T
_NC = 2
_NS = 16
_NW = _NC * _NS
_PER_W = _B // _NW
_CB = 4
_NBUF = 4
_NCHUNK = _PER_W // _CB
_NOUT = _NCHUNK // _NBUF


@functools.partial(
    pl.kernel,
    out_type=jax.ShapeDtypeStruct((_B, _T, _D), jnp.float32),
    mesh=plsc.VectorSubcoreMesh(core_axis_name="c", subcore_axis_name="s"),
    scratch_types=[
        pltpu.VMEM((_NBUF, _CB, _T), jnp.int32),
        pltpu.VMEM((_NBUF, _CB, _T, _D), jnp.float32),
    ] + [pltpu.SemaphoreType.DMA] * (2 * _NBUF),
    compiler_params=pltpu.CompilerParams(use_tc_tiling_on_sc=False),
)
def _gather_kernel(idx_hbm, table_hbm, out_hbm, idx_v, rows_v, *sems):
    gsem = sems[:_NBUF]
    ssem = sems[_NBUF:]
    wid = lax.axis_index("s") * _NC + lax.axis_index("c")
    base = wid * _PER_W

    def start_gathers(b):
        for i in range(_CB):
            pltpu.async_copy(table_hbm.at[idx_v.at[b].at[i]],
                             rows_v.at[b].at[i], gsem[b])

    def wait_gathers(b):
        for i in range(_CB):
            pltpu.make_async_copy(table_hbm.at[idx_v.at[b].at[i]],
                                  rows_v.at[b].at[i], gsem[b]).wait()

    for b in range(_NBUF):
        off = base + b * _CB
        pltpu.sync_copy(idx_hbm.at[pl.ds(off, _CB)], idx_v.at[b])
        start_gathers(b)

    def body(j, carry):
        for b in range(_NBUF):
            g = j * _NBUF + b
            off = base + g * _CB
            wait_gathers(b)
            pltpu.async_copy(rows_v.at[b], out_hbm.at[pl.ds(off, _CB)],
                             ssem[b])

            @pl.when(j < _NOUT - 1)
            def _():
                off2 = base + (g + _NBUF) * _CB
                pltpu.make_async_copy(
                    rows_v.at[b], out_hbm.at[pl.ds(off, _CB)], ssem[b]
                ).wait()
                pltpu.sync_copy(idx_hbm.at[pl.ds(off2, _CB)], idx_v.at[b])
                start_gathers(b)
        return carry

    lax.fori_loop(0, _NOUT, body, 0)

    for b in range(_NBUF):
        off = base + ((_NOUT - 1) * _NBUF + b) * _CB
        pltpu.make_async_copy(
            rows_v.at[b], out_hbm.at[pl.ds(off, _CB)], ssem[b]
        ).wait()


def kernel(x, weight):
    return _gather_kernel(x, weight)
